# baseline (speedup 1.0000x reference)
import jax
import jax.numpy as jnp
from jax import lax
from jax.experimental import pallas as pl
from jax.experimental.pallas import tpu as pltpu
from jax.experimental.pallas import tpu_sc as plsc

N = 10000
E = 320000
D = 128
H = 8
C = 16
ACC_W = 144
EB = 40
NTILES = 32
EDGES_PER_TILE = E // NTILES
NUM_BATCHES = EDGES_PER_TILE // EB
ROWS_PER_TILE = N // 16


def _lin_body(x_ref, wl_ref, bl_ref, wr_ref, br_ref, xl_ref, xr_ref):
    xb = x_ref[...]
    xl_ref[...] = jnp.dot(xb, wl_ref[...], preferred_element_type=jnp.float32) + bl_ref[...]
    xr_ref[...] = jnp.dot(xb, wr_ref[...], preferred_element_type=jnp.float32) + br_ref[...]


def _linear(xp, Wl, bl, Wr, br):
    BN = 1000
    return pl.pallas_call(
        _lin_body,
        grid=(N // BN,),
        in_specs=[
            pl.BlockSpec((BN, D), lambda i: (i, 0)),
            pl.BlockSpec((D, D), lambda i: (0, 0)),
            pl.BlockSpec((1, D), lambda i: (0, 0)),
            pl.BlockSpec((D, D), lambda i: (0, 0)),
            pl.BlockSpec((1, D), lambda i: (0, 0)),
        ],
        out_specs=[
            pl.BlockSpec((BN, D), lambda i: (i, 0)),
            pl.BlockSpec((BN, D), lambda i: (i, 0)),
        ],
        out_shape=[jax.ShapeDtypeStruct((N, D), jnp.float32)] * 2,
    )(xp, Wl, bl.reshape(1, D), Wr, br.reshape(1, D))


_GDN = lax.GatherDimensionNumbers(
    offset_dims=(), collapsed_slice_dims=(0,), start_index_map=(0,))


def _lane_shuffle(v, perm):
    return lax.gather(v, perm[:, None], _GDN, slice_sizes=(1,),
                      mode=lax.GatherScatterMode.PROMISE_IN_BOUNDS)


def _sum_splat(v, perms):
    for perm in perms:
        v = v + _lane_shuffle(v, perm)
    return v


def _edge_body(xl_hbm, xr_hbm, src_hbm, dst_hbm, att_hbm, out_hbm,
               accum, srcg_a, srcg_b, dstg_a, dstg_b, dsts_a, dsts_b,
               xl_a, xr_a, xl_b, xr_b, orow_a, orow_b, att_v,
               sem_ga, sem_gb, sem_da, sem_db, sem_sa, sem_sb):
    c = lax.axis_index("c")
    s = lax.axis_index("s")
    wid = c * 16 + s
    base0 = wid * EDGES_PER_TILE

    pltpu.sync_copy(att_hbm, att_v)

    zero = jnp.zeros((16,), jnp.float32)

    def zrow(i, carry):
        for j in range(ACC_W // 16):
            orow_a[i, pl.ds(j * 16, 16)] = zero
            orow_b[i, pl.ds(j * 16, 16)] = zero
        return carry

    lax.fori_loop(0, EB, zrow, 0)
    for k in range(ROWS_PER_TILE // EB):
        pltpu.sync_copy(orow_a, accum.at[pl.ds(s * ROWS_PER_TILE + k * EB, EB)])
    rem = ROWS_PER_TILE % EB
    if rem:
        pltpu.sync_copy(
            orow_a.at[pl.ds(0, rem)],
            accum.at[pl.ds(s * ROWS_PER_TILE + (ROWS_PER_TILE // EB) * EB, rem)])
    plsc.subcore_barrier()

    att_regs = [att_v[h, :] for h in range(H)]
    lane = lax.iota(jnp.int32, 16)
    perms = [lane ^ sh for sh in (1, 2, 4, 8)]
    odd_pair = (lane & 1) == 1
    hi_quad = (lane & 2) == 2
    hi_oct = (lane & 4) == 4

    last = NUM_BATCHES - 1

    def issue_gidx(b, srcg, dstg, sem_d):
        pltpu.async_copy(src_hbm.at[pl.ds(base0 + b * EB, EB)], srcg, sem_d)
        pltpu.async_copy(dst_hbm.at[pl.ds(base0 + b * EB, EB)], dstg, sem_d)

    def wait_gidx(srcg, dstg, sem_d):
        pltpu.make_async_copy(src_hbm.at[pl.ds(0, EB)], srcg, sem_d).wait()
        pltpu.make_async_copy(dst_hbm.at[pl.ds(0, EB)], dstg, sem_d).wait()

    def issue_gathers(xl_buf, xr_buf, srcg, dstg, sem_g):
        pltpu.async_copy(xl_hbm.at[srcg], xl_buf, sem_g)
        pltpu.async_copy(xr_hbm.at[dstg], xr_buf, sem_g)

    def wait_gathers(xl_buf, xr_buf, sem_g):
        pltpu.make_async_copy(src_hbm.at[pl.ds(0, EB)], xl_buf, sem_g).wait()
        pltpu.make_async_copy(src_hbm.at[pl.ds(0, EB)], xr_buf, sem_g).wait()

    def vcopy_idx(src_ref, dst_ref):
        for off in (0, 16, EB - 16):
            dst_ref[pl.ds(off, 16)] = src_ref[pl.ds(off, 16)]

    def issue_scatter(orow, dsts, sem_s):
        pltpu.async_copy(orow, accum.at[dsts], sem_s, add=True)

    def wait_scatter(orow, dsts, sem_s):
        pltpu.make_async_copy(orow, accum.at[dsts], sem_s).wait()

    def compute(xl_buf, xr_buf, orow):
        @plsc.parallel_loop(0, EB, unroll=4)
        def edge_body(i):
            evs = []
            for h in range(H):
                xj = xl_buf[i, pl.ds(h * 16, 16)]
                xi = xr_buf[i, pl.ds(h * 16, 16)]
                sv = xi + xj
                t = jnp.maximum(sv, 0.2 * sv)
                alpha = _sum_splat(t * att_regs[h], perms)
                ev = jnp.exp(alpha)
                orow[i, pl.ds(h * 16, 16)] = xj * ev
                evs.append(ev)
            p01 = jnp.where(odd_pair, evs[1], evs[0])
            p23 = jnp.where(odd_pair, evs[3], evs[2])
            p45 = jnp.where(odd_pair, evs[5], evs[4])
            p67 = jnp.where(odd_pair, evs[7], evs[6])
            q03 = jnp.where(hi_quad, p23, p01)
            q47 = jnp.where(hi_quad, p67, p45)
            orow[i, pl.ds(D, 16)] = jnp.where(hi_oct, q47, q03)

    A = (xl_a, xr_a, srcg_a, dstg_a, dsts_a, orow_a, sem_ga, sem_da, sem_sa)
    B = (xl_b, xr_b, srcg_b, dstg_b, dsts_b, orow_b, sem_gb, sem_db, sem_sb)

    issue_gidx(0, srcg_a, dstg_a, sem_da)
    wait_gidx(srcg_a, dstg_a, sem_da)
    issue_gathers(xl_a, xr_a, srcg_a, dstg_a, sem_ga)
    issue_gidx(1, srcg_b, dstg_b, sem_db)
    vcopy_idx(dstg_a, dsts_a)
    vcopy_idx(dstg_a, dsts_b)
    issue_scatter(orow_a, dsts_a, sem_sa)
    issue_scatter(orow_b, dsts_b, sem_sb)

    def batch_pair(k, carry):
        for par, (P, Q) in ((0, (A, B)), (1, (B, A))):
            b = 2 * k + par
            xlP, xrP, srcgP, dstgP, dstsP, orowP, sem_gP, sem_dP, sem_sP = P
            xlQ, xrQ, srcgQ, dstgQ, dstsQ, orowQ, sem_gQ, sem_dQ, sem_sQ = Q
            wait_gidx(srcgQ, dstgQ, sem_dQ)
            issue_gathers(xlQ, xrQ, srcgQ, dstgQ, sem_gQ)
            wait_gathers(xlP, xrP, sem_gP)
            wait_scatter(orowP, dstsP, sem_sP)
            vcopy_idx(dstgP, dstsP)
            issue_gidx(jnp.minimum(b + 2, last), srcgP, dstgP, sem_dP)
            compute(xlP, xrP, orowP)
            issue_scatter(orowP, dstsP, sem_sP)
        return carry

    lax.fori_loop(0, NUM_BATCHES // 2, batch_pair, 0)
    wait_scatter(orow_a, dsts_a, sem_sa)
    wait_scatter(orow_b, dsts_b, sem_sb)
    wait_gathers(xl_a, xr_a, sem_ga)
    wait_gidx(srcg_b, dstg_b, sem_db)
    plsc.subcore_barrier()

    pltpu.sync_copy(
        accum.at[pl.ds(s * ROWS_PER_TILE, ROWS_PER_TILE)],
        out_hbm.at[pl.ds(c * N + s * ROWS_PER_TILE, ROWS_PER_TILE)],
    )


def _edge_pass(xl, xr, src, dst, att):
    mesh = plsc.VectorSubcoreMesh(core_axis_name="c", subcore_axis_name="s")
    kfn = pl.kernel(
        _edge_body,
        out_type=jax.ShapeDtypeStruct((2 * N, ACC_W), jnp.float32),
        mesh=mesh,
        scratch_types=[
            pltpu.VMEM_SHARED((N, ACC_W), jnp.float32),
            pltpu.VMEM((EB,), jnp.int32),
            pltpu.VMEM((EB,), jnp.int32),
            pltpu.VMEM((EB,), jnp.int32),
            pltpu.VMEM((EB,), jnp.int32),
            pltpu.VMEM((EB,), jnp.int32),
            pltpu.VMEM((EB,), jnp.int32),
            pltpu.VMEM((EB, D), jnp.float32),
            pltpu.VMEM((EB, D), jnp.float32),
            pltpu.VMEM((EB, D), jnp.float32),
            pltpu.VMEM((EB, D), jnp.float32),
            pltpu.VMEM((EB, ACC_W), jnp.float32),
            pltpu.VMEM((EB, ACC_W), jnp.float32),
            pltpu.VMEM((H, C), jnp.float32),
            pltpu.SemaphoreType.DMA,
            pltpu.SemaphoreType.DMA,
            pltpu.SemaphoreType.DMA,
            pltpu.SemaphoreType.DMA,
            pltpu.SemaphoreType.DMA,
            pltpu.SemaphoreType.DMA,
        ],
        compiler_params=pltpu.CompilerParams(use_tc_tiling_on_sc=False),
    )
    return kfn(xl, xr, src, dst, att)


def _final_body(p0_ref, p1_ref, x_ref, bias_ref, gamma_ref, beta_ref, o_ref):
    num = p0_ref[:, :D] + p1_ref[:, :D]
    den16 = p0_ref[:, D:] + p1_ref[:, D:]
    eh = lax.broadcasted_iota(jnp.int32, (16, D), 0)
    ej = lax.broadcasted_iota(jnp.int32, (16, D), 1) // C
    expand = (eh == ej).astype(jnp.float32)
    den = jnp.dot(den16, expand, preferred_element_type=jnp.float32)
    out = num / (den + 1e-16)
    hh = out + bias_ref[...]
    hh = jnp.where(hh > 0, hh, jnp.exp(jnp.minimum(hh, 0.0)) - 1.0)
    y = hh + x_ref[...]
    mu = jnp.mean(y, axis=-1, keepdims=True)
    var = jnp.mean((y - mu) ** 2, axis=-1, keepdims=True)
    o_ref[...] = gamma_ref[...] * (y - mu) / jnp.sqrt(var + 1e-5) + beta_ref[...]


def _finalize(partials, xp, bias, gamma, beta):
    BN = 1000
    nblk = N // BN
    return pl.pallas_call(
        _final_body,
        grid=(nblk,),
        in_specs=[
            pl.BlockSpec((BN, ACC_W), lambda i: (i, 0)),
            pl.BlockSpec((BN, ACC_W), lambda i, _n=nblk: (i + _n, 0)),
            pl.BlockSpec((BN, D), lambda i: (i, 0)),
            pl.BlockSpec((1, D), lambda i: (0, 0)),
            pl.BlockSpec((1, D), lambda i: (0, 0)),
            pl.BlockSpec((1, D), lambda i: (0, 0)),
        ],
        out_specs=pl.BlockSpec((BN, D), lambda i: (i, 0)),
        out_shape=jax.ShapeDtypeStruct((N, D), jnp.float32),
    )(partials, partials, xp, bias.reshape(1, D), gamma.reshape(1, D), beta.reshape(1, D))


def kernel(x, edge_index,
           Wl0, bl0, Wr0, br0, att0, bias0, gamma0, beta0,
           Wl1, bl1, Wr1, br1, att1, bias1, gamma1, beta1):
    src = edge_index[0].astype(jnp.int32)
    dst = edge_index[1].astype(jnp.int32)
    xp = x
    layers = (
        (Wl0, bl0, Wr0, br0, att0, bias0, gamma0, beta0),
        (Wl1, bl1, Wr1, br1, att1, bias1, gamma1, beta1),
    )
    for (Wl, bl, Wr, br, att, bias, gamma, beta) in layers:
        xl, xr = _linear(xp, Wl, bl, Wr, br)
        partials = _edge_pass(xl, xr, src, dst, att)
        xp = _finalize(partials, xp, bias, gamma, beta)
    return xp[:N]


# speedup vs baseline: 1.4152x; 1.4152x over previous
import jax
import jax.numpy as jnp
import numpy as np
from jax import lax
from jax.experimental import pallas as pl
from jax.experimental.pallas import tpu as pltpu
from jax.experimental.pallas import tpu_sc as plsc

N = 10000
E = 320000
D = 128
H = 8
C = 16
ACC_W = 144
EB = 40
NTILES = 32
EDGES_PER_TILE = E // NTILES
NUM_BATCHES = EDGES_PER_TILE // EB
ROWS_PER_TILE = N // 16

_PERM = np.empty((D,), np.int32)
for _p in range(H // 2):
    for _c in range(C):
        _PERM[32 * _p + 2 * _c] = (2 * _p) * 16 + _c
        _PERM[32 * _p + 2 * _c + 1] = (2 * _p + 1) * 16 + _c


def _lin_body(x_ref, wl_ref, bl_ref, wr_ref, br_ref, xl_ref, xr_ref):
    xb = x_ref[...]
    xl_ref[...] = (jnp.dot(xb, wl_ref[...], preferred_element_type=jnp.float32)
                   + bl_ref[...]).astype(jnp.bfloat16)
    xr_ref[...] = (jnp.dot(xb, wr_ref[...], preferred_element_type=jnp.float32)
                   + br_ref[...]).astype(jnp.bfloat16)


def _linear(xp, Wl, bl, Wr, br):
    BN = 1000
    return pl.pallas_call(
        _lin_body,
        grid=(N // BN,),
        in_specs=[
            pl.BlockSpec((BN, D), lambda i: (i, 0)),
            pl.BlockSpec((D, D), lambda i: (0, 0)),
            pl.BlockSpec((1, D), lambda i: (0, 0)),
            pl.BlockSpec((D, D), lambda i: (0, 0)),
            pl.BlockSpec((1, D), lambda i: (0, 0)),
        ],
        out_specs=[
            pl.BlockSpec((BN, D), lambda i: (i, 0)),
            pl.BlockSpec((BN, D), lambda i: (i, 0)),
        ],
        out_shape=[jax.ShapeDtypeStruct((N, D), jnp.bfloat16)] * 2,
    )(xp, Wl, bl.reshape(1, D), Wr, br.reshape(1, D))


_GDN = lax.GatherDimensionNumbers(
    offset_dims=(), collapsed_slice_dims=(0,), start_index_map=(0,))


def _lane_shuffle(v, perm):
    return lax.gather(v, perm[:, None], _GDN, slice_sizes=(1,),
                      mode=lax.GatherScatterMode.PROMISE_IN_BOUNDS)


def _sum_splat(v, perms):
    for perm in perms:
        v = v + _lane_shuffle(v, perm)
    return v


def _edge_body(xl_hbm, xr_hbm, src_hbm, dst_hbm, att_hbm, out_hbm,
               accum, srcg_a, srcg_b, dstg_a, dstg_b, dsts_a, dsts_b,
               xl_a, xr_a, xl_b, xr_b, orow_a, orow_b, att_v,
               sem_ga, sem_gb, sem_da, sem_db, sem_sa, sem_sb):
    c = lax.axis_index("c")
    s = lax.axis_index("s")
    wid = c * 16 + s
    base0 = wid * EDGES_PER_TILE

    pltpu.sync_copy(att_hbm, att_v)

    zero = jnp.zeros((16,), jnp.float32)

    def zrow(i, carry):
        for j in range(ACC_W // 16):
            orow_a[i, pl.ds(j * 16, 16)] = zero
            orow_b[i, pl.ds(j * 16, 16)] = zero
        return carry

    lax.fori_loop(0, EB, zrow, 0)
    for k in range(ROWS_PER_TILE // EB):
        pltpu.sync_copy(orow_a, accum.at[pl.ds(s * ROWS_PER_TILE + k * EB, EB)])
    rem = ROWS_PER_TILE % EB
    if rem:
        pltpu.sync_copy(
            orow_a.at[pl.ds(0, rem)],
            accum.at[pl.ds(s * ROWS_PER_TILE + (ROWS_PER_TILE // EB) * EB, rem)])
    plsc.subcore_barrier()

    att_regs = [att_v[h, :] for h in range(H)]
    lane = lax.iota(jnp.int32, 16)
    perms = [lane ^ sh for sh in (1, 2, 4, 8)]

    last = NUM_BATCHES - 1

    def issue_gidx(b, srcg, dstg, sem_d):
        pltpu.async_copy(src_hbm.at[pl.ds(base0 + b * EB, EB)], srcg, sem_d)
        pltpu.async_copy(dst_hbm.at[pl.ds(base0 + b * EB, EB)], dstg, sem_d)

    def wait_gidx(srcg, dstg, sem_d):
        pltpu.make_async_copy(src_hbm.at[pl.ds(0, EB)], srcg, sem_d).wait()
        pltpu.make_async_copy(dst_hbm.at[pl.ds(0, EB)], dstg, sem_d).wait()

    def issue_gathers(xl_buf, xr_buf, srcg, dstg, sem_g):
        pltpu.async_copy(xl_hbm.at[srcg], xl_buf, sem_g)
        pltpu.async_copy(xr_hbm.at[dstg], xr_buf, sem_g)

    def wait_gathers(xl_buf, xr_buf, sem_g):
        pltpu.make_async_copy(src_hbm.at[pl.ds(0, EB)], xl_buf, sem_g).wait()
        pltpu.make_async_copy(src_hbm.at[pl.ds(0, EB)], xr_buf, sem_g).wait()

    def vcopy_idx(src_ref, dst_ref):
        for off in (0, 16, EB - 16):
            dst_ref[pl.ds(off, 16)] = src_ref[pl.ds(off, 16)]

    def issue_scatter(orow, dsts, sem_s):
        pltpu.async_copy(orow, accum.at[dsts], sem_s, add=True)

    def wait_scatter(orow, dsts, sem_s):
        pltpu.make_async_copy(orow, accum.at[dsts], sem_s).wait()

    def compute(xl_buf, xr_buf, orow):
        @plsc.parallel_loop(0, EB, unroll=4)
        def edge_body(i):
            dv = jnp.zeros((16,), jnp.float32)
            for p in range(H // 2):
                xjs = plsc.unpack(xl_buf[i, pl.ds(p * 32, 32)],
                                  format=plsc.PackFormat.INTERLEAVED)
                xis = plsc.unpack(xr_buf[i, pl.ds(p * 32, 32)],
                                  format=plsc.PackFormat.INTERLEAVED)
                for q in (0, 1):
                    h = 2 * p + q
                    xj = xjs[q]
                    sv = xis[q] + xj
                    t = jnp.maximum(sv, 0.2 * sv)
                    alpha = _sum_splat(t * att_regs[h], perms)
                    ev = jnp.exp(alpha)
                    orow[i, pl.ds(h * 16, 16)] = xj * ev
                    dv = jnp.where(lane == h, ev, dv)
            orow[i, pl.ds(D, 16)] = dv

    A = (xl_a, xr_a, srcg_a, dstg_a, dsts_a, orow_a, sem_ga, sem_da, sem_sa)
    B = (xl_b, xr_b, srcg_b, dstg_b, dsts_b, orow_b, sem_gb, sem_db, sem_sb)

    issue_gidx(0, srcg_a, dstg_a, sem_da)
    wait_gidx(srcg_a, dstg_a, sem_da)
    issue_gathers(xl_a, xr_a, srcg_a, dstg_a, sem_ga)
    issue_gidx(1, srcg_b, dstg_b, sem_db)
    vcopy_idx(dstg_a, dsts_a)
    vcopy_idx(dstg_a, dsts_b)
    issue_scatter(orow_a, dsts_a, sem_sa)
    issue_scatter(orow_b, dsts_b, sem_sb)

    def batch_pair(k, carry):
        for par, (P, Q) in ((0, (A, B)), (1, (B, A))):
            b = 2 * k + par
            xlP, xrP, srcgP, dstgP, dstsP, orowP, sem_gP, sem_dP, sem_sP = P
            xlQ, xrQ, srcgQ, dstgQ, dstsQ, orowQ, sem_gQ, sem_dQ, sem_sQ = Q
            wait_gidx(srcgQ, dstgQ, sem_dQ)
            issue_gathers(xlQ, xrQ, srcgQ, dstgQ, sem_gQ)
            wait_gathers(xlP, xrP, sem_gP)
            wait_scatter(orowP, dstsP, sem_sP)
            vcopy_idx(dstgP, dstsP)
            issue_gidx(jnp.minimum(b + 2, last), srcgP, dstgP, sem_dP)
            compute(xlP, xrP, orowP)
            issue_scatter(orowP, dstsP, sem_sP)
        return carry

    lax.fori_loop(0, NUM_BATCHES // 2, batch_pair, 0)
    wait_scatter(orow_a, dsts_a, sem_sa)
    wait_scatter(orow_b, dsts_b, sem_sb)
    wait_gathers(xl_a, xr_a, sem_ga)
    wait_gidx(srcg_b, dstg_b, sem_db)
    plsc.subcore_barrier()

    pltpu.sync_copy(
        accum.at[pl.ds(s * ROWS_PER_TILE, ROWS_PER_TILE)],
        out_hbm.at[pl.ds(c * N + s * ROWS_PER_TILE, ROWS_PER_TILE)],
    )


def _edge_pass(xl, xr, src, dst, att):
    mesh = plsc.VectorSubcoreMesh(core_axis_name="c", subcore_axis_name="s")
    kfn = pl.kernel(
        _edge_body,
        out_type=jax.ShapeDtypeStruct((2 * N, ACC_W), jnp.float32),
        mesh=mesh,
        scratch_types=[
            pltpu.VMEM_SHARED((N, ACC_W), jnp.float32),
            pltpu.VMEM((EB,), jnp.int32),
            pltpu.VMEM((EB,), jnp.int32),
            pltpu.VMEM((EB,), jnp.int32),
            pltpu.VMEM((EB,), jnp.int32),
            pltpu.VMEM((EB,), jnp.int32),
            pltpu.VMEM((EB,), jnp.int32),
            pltpu.VMEM((EB, D), jnp.bfloat16),
            pltpu.VMEM((EB, D), jnp.bfloat16),
            pltpu.VMEM((EB, D), jnp.bfloat16),
            pltpu.VMEM((EB, D), jnp.bfloat16),
            pltpu.VMEM((EB, ACC_W), jnp.float32),
            pltpu.VMEM((EB, ACC_W), jnp.float32),
            pltpu.VMEM((H, C), jnp.float32),
            pltpu.SemaphoreType.DMA,
            pltpu.SemaphoreType.DMA,
            pltpu.SemaphoreType.DMA,
            pltpu.SemaphoreType.DMA,
            pltpu.SemaphoreType.DMA,
            pltpu.SemaphoreType.DMA,
        ],
        compiler_params=pltpu.CompilerParams(use_tc_tiling_on_sc=False,
                                             needs_layout_passes=False),
    )
    return kfn(xl, xr, src, dst, att)


def _final_body(p0_ref, p1_ref, x_ref, bias_ref, gamma_ref, beta_ref, o_ref):
    num = p0_ref[:, :D] + p1_ref[:, :D]
    den16 = p0_ref[:, D:] + p1_ref[:, D:]
    eh = lax.broadcasted_iota(jnp.int32, (16, D), 0)
    ej = lax.broadcasted_iota(jnp.int32, (16, D), 1) // C
    expand = (eh == ej).astype(jnp.float32)
    den = jnp.dot(den16, expand, preferred_element_type=jnp.float32)
    out = num / (den + 1e-16)
    hh = out + bias_ref[...]
    hh = jnp.where(hh > 0, hh, jnp.exp(jnp.minimum(hh, 0.0)) - 1.0)
    y = hh + x_ref[...]
    mu = jnp.mean(y, axis=-1, keepdims=True)
    var = jnp.mean((y - mu) ** 2, axis=-1, keepdims=True)
    o_ref[...] = gamma_ref[...] * (y - mu) / jnp.sqrt(var + 1e-5) + beta_ref[...]


def _finalize(partials, xp, bias, gamma, beta):
    BN = 1000
    nblk = N // BN
    return pl.pallas_call(
        _final_body,
        grid=(nblk,),
        in_specs=[
            pl.BlockSpec((BN, ACC_W), lambda i: (i, 0)),
            pl.BlockSpec((BN, ACC_W), lambda i, _n=nblk: (i + _n, 0)),
            pl.BlockSpec((BN, D), lambda i: (i, 0)),
            pl.BlockSpec((1, D), lambda i: (0, 0)),
            pl.BlockSpec((1, D), lambda i: (0, 0)),
            pl.BlockSpec((1, D), lambda i: (0, 0)),
        ],
        out_specs=pl.BlockSpec((BN, D), lambda i: (i, 0)),
        out_shape=jax.ShapeDtypeStruct((N, D), jnp.float32),
    )(partials, partials, xp, bias.reshape(1, D), gamma.reshape(1, D), beta.reshape(1, D))


def kernel(x, edge_index,
           Wl0, bl0, Wr0, br0, att0, bias0, gamma0, beta0,
           Wl1, bl1, Wr1, br1, att1, bias1, gamma1, beta1):
    src = edge_index[0].astype(jnp.int32)
    dst = edge_index[1].astype(jnp.int32)
    xp = x
    layers = (
        (Wl0, bl0, Wr0, br0, att0, bias0, gamma0, beta0),
        (Wl1, bl1, Wr1, br1, att1, bias1, gamma1, beta1),
    )
    for (Wl, bl, Wr, br, att, bias, gamma, beta) in layers:
        xl, xr = _linear(xp, Wl[:, _PERM], bl[_PERM], Wr[:, _PERM], br[_PERM])
        partials = _edge_pass(xl, xr, src, dst, att)
        xp = _finalize(partials, xp, bias, gamma, beta)
    return xp[:N]


# speedup vs baseline: 1.7977x; 1.2703x over previous
import jax
import jax.numpy as jnp
import numpy as np
from jax import lax
from jax.experimental import pallas as pl
from jax.experimental.pallas import tpu as pltpu
from jax.experimental.pallas import tpu_sc as plsc

N = 10000
E = 320000
D = 128
H = 8
C = 16
ACC_W = 144
EB = 40
NTILES = 32
EDGES_PER_TILE = E // NTILES
NUM_BATCHES = EDGES_PER_TILE // EB
ROWS_PER_TILE = N // 16

_PERM = np.empty((D,), np.int32)
for _p in range(H // 2):
    for _c in range(C):
        _PERM[32 * _p + 2 * _c] = (2 * _p) * 16 + _c
        _PERM[32 * _p + 2 * _c + 1] = (2 * _p + 1) * 16 + _c


def _lin_body(x_ref, wl_ref, bl_ref, wr_ref, br_ref, xl_ref, xr_ref):
    xb = x_ref[...]
    xl_ref[...] = (jnp.dot(xb, wl_ref[...], preferred_element_type=jnp.float32)
                   + bl_ref[...]).astype(jnp.bfloat16)
    xr_ref[...] = (jnp.dot(xb, wr_ref[...], preferred_element_type=jnp.float32)
                   + br_ref[...]).astype(jnp.bfloat16)


def _linear(xp, Wl, bl, Wr, br):
    BN = 1000
    return pl.pallas_call(
        _lin_body,
        grid=(N // BN,),
        in_specs=[
            pl.BlockSpec((BN, D), lambda i: (i, 0)),
            pl.BlockSpec((D, D), lambda i: (0, 0)),
            pl.BlockSpec((1, D), lambda i: (0, 0)),
            pl.BlockSpec((D, D), lambda i: (0, 0)),
            pl.BlockSpec((1, D), lambda i: (0, 0)),
        ],
        out_specs=[
            pl.BlockSpec((BN, D), lambda i: (i, 0)),
            pl.BlockSpec((BN, D), lambda i: (i, 0)),
        ],
        out_shape=[jax.ShapeDtypeStruct((N, D), jnp.bfloat16)] * 2,
    )(xp, Wl, bl.reshape(1, D), Wr, br.reshape(1, D))


_GDN = lax.GatherDimensionNumbers(
    offset_dims=(), collapsed_slice_dims=(0,), start_index_map=(0,))


def _lane_shuffle(v, perm):
    return lax.gather(v, perm[:, None], _GDN, slice_sizes=(1,),
                      mode=lax.GatherScatterMode.PROMISE_IN_BOUNDS)


def _sum_splat(v, perms):
    for perm in perms:
        v = v + _lane_shuffle(v, perm)
    return v


def _edge_body(xl_hbm, xr_hbm, src_hbm, dst_hbm, att_hbm, out_hbm,
               accum, srcg_a, srcg_b, dstg_a, dstg_b, dsts_a, dsts_b,
               xl_a, xr_a, xl_b, xr_b, orow_a, orow_b, att_v,
               sem_ga, sem_gb, sem_da, sem_db, sem_sa, sem_sb):
    c = lax.axis_index("c")
    s = lax.axis_index("s")
    wid = c * 16 + s
    base0 = wid * EDGES_PER_TILE

    pltpu.sync_copy(att_hbm, att_v)

    zero = jnp.zeros((16,), jnp.float32)

    def zrow(i, carry):
        for j in range(ACC_W // 16):
            orow_a[i, pl.ds(j * 16, 16)] = zero
            orow_b[i, pl.ds(j * 16, 16)] = zero
        return carry

    lax.fori_loop(0, EB, zrow, 0)
    for k in range(ROWS_PER_TILE // EB):
        pltpu.sync_copy(orow_a, accum.at[pl.ds(s * ROWS_PER_TILE + k * EB, EB)])
    rem = ROWS_PER_TILE % EB
    if rem:
        pltpu.sync_copy(
            orow_a.at[pl.ds(0, rem)],
            accum.at[pl.ds(s * ROWS_PER_TILE + (ROWS_PER_TILE // EB) * EB, rem)])
    plsc.subcore_barrier()

    att_regs = [att_v[h, :] for h in range(H)]
    lane = lax.iota(jnp.int32, 16)
    perm15 = lane * 0 + 15

    last = NUM_BATCHES - 1

    def issue_gidx(b, srcg, dstg, sem_d):
        pltpu.async_copy(src_hbm.at[pl.ds(base0 + b * EB, EB)], srcg, sem_d)
        pltpu.async_copy(dst_hbm.at[pl.ds(base0 + b * EB, EB)], dstg, sem_d)

    def wait_gidx(srcg, dstg, sem_d):
        pltpu.make_async_copy(src_hbm.at[pl.ds(0, EB)], srcg, sem_d).wait()
        pltpu.make_async_copy(dst_hbm.at[pl.ds(0, EB)], dstg, sem_d).wait()

    def issue_gathers(xl_buf, xr_buf, srcg, dstg, sem_g):
        pltpu.async_copy(xl_hbm.at[srcg], xl_buf, sem_g)
        pltpu.async_copy(xr_hbm.at[dstg], xr_buf, sem_g)

    def wait_gathers(xl_buf, xr_buf, sem_g):
        pltpu.make_async_copy(src_hbm.at[pl.ds(0, EB)], xl_buf, sem_g).wait()
        pltpu.make_async_copy(src_hbm.at[pl.ds(0, EB)], xr_buf, sem_g).wait()

    def vcopy_idx(src_ref, dst_ref):
        for off in (0, 16, EB - 16):
            dst_ref[pl.ds(off, 16)] = src_ref[pl.ds(off, 16)]

    def issue_scatter(orow, dsts, sem_s):
        pltpu.async_copy(orow, accum.at[dsts], sem_s, add=True)

    def wait_scatter(orow, dsts, sem_s):
        pltpu.make_async_copy(orow, accum.at[dsts], sem_s).wait()

    def compute(xl_buf, xr_buf, orow):
        @plsc.parallel_loop(0, EB, unroll=4)
        def edge_body(i):
            dv = jnp.zeros((16,), jnp.float32)
            for p in range(H // 2):
                xjs = plsc.unpack(xl_buf[i, pl.ds(p * 32, 32)],
                                  format=plsc.PackFormat.INTERLEAVED)
                xis = plsc.unpack(xr_buf[i, pl.ds(p * 32, 32)],
                                  format=plsc.PackFormat.INTERLEAVED)
                for q in (0, 1):
                    h = 2 * p + q
                    xj = xjs[q]
                    sv = xis[q] + xj
                    t = jnp.maximum(sv, 0.2 * sv)
                    alpha = _lane_shuffle(jnp.cumsum(t * att_regs[h]), perm15)
                    ev = jnp.exp(alpha)
                    orow[i, pl.ds(h * 16, 16)] = xj * ev
                    dv = jnp.where(lane == h, ev, dv)
            orow[i, pl.ds(D, 16)] = dv

    A = (xl_a, xr_a, srcg_a, dstg_a, dsts_a, orow_a, sem_ga, sem_da, sem_sa)
    B = (xl_b, xr_b, srcg_b, dstg_b, dsts_b, orow_b, sem_gb, sem_db, sem_sb)

    issue_gidx(0, srcg_a, dstg_a, sem_da)
    wait_gidx(srcg_a, dstg_a, sem_da)
    issue_gathers(xl_a, xr_a, srcg_a, dstg_a, sem_ga)
    issue_gidx(1, srcg_b, dstg_b, sem_db)
    vcopy_idx(dstg_a, dsts_a)
    vcopy_idx(dstg_a, dsts_b)
    issue_scatter(orow_a, dsts_a, sem_sa)
    issue_scatter(orow_b, dsts_b, sem_sb)

    def batch_pair(k, carry):
        for par, (P, Q) in ((0, (A, B)), (1, (B, A))):
            b = 2 * k + par
            xlP, xrP, srcgP, dstgP, dstsP, orowP, sem_gP, sem_dP, sem_sP = P
            xlQ, xrQ, srcgQ, dstgQ, dstsQ, orowQ, sem_gQ, sem_dQ, sem_sQ = Q
            wait_gidx(srcgQ, dstgQ, sem_dQ)
            issue_gathers(xlQ, xrQ, srcgQ, dstgQ, sem_gQ)
            wait_gathers(xlP, xrP, sem_gP)
            wait_scatter(orowP, dstsP, sem_sP)
            vcopy_idx(dstgP, dstsP)
            issue_gidx(jnp.minimum(b + 2, last), srcgP, dstgP, sem_dP)
            compute(xlP, xrP, orowP)
            issue_scatter(orowP, dstsP, sem_sP)
        return carry

    lax.fori_loop(0, NUM_BATCHES // 2, batch_pair, 0)
    wait_scatter(orow_a, dsts_a, sem_sa)
    wait_scatter(orow_b, dsts_b, sem_sb)
    wait_gathers(xl_a, xr_a, sem_ga)
    wait_gidx(srcg_b, dstg_b, sem_db)
    plsc.subcore_barrier()

    pltpu.sync_copy(
        accum.at[pl.ds(s * ROWS_PER_TILE, ROWS_PER_TILE)],
        out_hbm.at[pl.ds(c * N + s * ROWS_PER_TILE, ROWS_PER_TILE)],
    )


def _edge_pass(xl, xr, src, dst, att):
    mesh = plsc.VectorSubcoreMesh(core_axis_name="c", subcore_axis_name="s")
    kfn = pl.kernel(
        _edge_body,
        out_type=jax.ShapeDtypeStruct((2 * N, ACC_W), jnp.float32),
        mesh=mesh,
        scratch_types=[
            pltpu.VMEM_SHARED((N, ACC_W), jnp.float32),
            pltpu.VMEM((EB,), jnp.int32),
            pltpu.VMEM((EB,), jnp.int32),
            pltpu.VMEM((EB,), jnp.int32),
            pltpu.VMEM((EB,), jnp.int32),
            pltpu.VMEM((EB,), jnp.int32),
            pltpu.VMEM((EB,), jnp.int32),
            pltpu.VMEM((EB, D), jnp.bfloat16),
            pltpu.VMEM((EB, D), jnp.bfloat16),
            pltpu.VMEM((EB, D), jnp.bfloat16),
            pltpu.VMEM((EB, D), jnp.bfloat16),
            pltpu.VMEM((EB, ACC_W), jnp.float32),
            pltpu.VMEM((EB, ACC_W), jnp.float32),
            pltpu.VMEM((H, C), jnp.float32),
            pltpu.SemaphoreType.DMA,
            pltpu.SemaphoreType.DMA,
            pltpu.SemaphoreType.DMA,
            pltpu.SemaphoreType.DMA,
            pltpu.SemaphoreType.DMA,
            pltpu.SemaphoreType.DMA,
        ],
        compiler_params=pltpu.CompilerParams(use_tc_tiling_on_sc=False,
                                             needs_layout_passes=False),
    )
    return kfn(xl, xr, src, dst, att)


def _final_body(p0_ref, p1_ref, x_ref, bias_ref, gamma_ref, beta_ref, o_ref):
    num = p0_ref[:, :D] + p1_ref[:, :D]
    den16 = p0_ref[:, D:] + p1_ref[:, D:]
    eh = lax.broadcasted_iota(jnp.int32, (16, D), 0)
    ej = lax.broadcasted_iota(jnp.int32, (16, D), 1) // C
    expand = (eh == ej).astype(jnp.float32)
    den = jnp.dot(den16, expand, preferred_element_type=jnp.float32)
    out = num / (den + 1e-16)
    hh = out + bias_ref[...]
    hh = jnp.where(hh > 0, hh, jnp.exp(jnp.minimum(hh, 0.0)) - 1.0)
    y = hh + x_ref[...]
    mu = jnp.mean(y, axis=-1, keepdims=True)
    var = jnp.mean((y - mu) ** 2, axis=-1, keepdims=True)
    o_ref[...] = gamma_ref[...] * (y - mu) / jnp.sqrt(var + 1e-5) + beta_ref[...]


def _finalize(partials, xp, bias, gamma, beta):
    BN = 1000
    nblk = N // BN
    return pl.pallas_call(
        _final_body,
        grid=(nblk,),
        in_specs=[
            pl.BlockSpec((BN, ACC_W), lambda i: (i, 0)),
            pl.BlockSpec((BN, ACC_W), lambda i, _n=nblk: (i + _n, 0)),
            pl.BlockSpec((BN, D), lambda i: (i, 0)),
            pl.BlockSpec((1, D), lambda i: (0, 0)),
            pl.BlockSpec((1, D), lambda i: (0, 0)),
            pl.BlockSpec((1, D), lambda i: (0, 0)),
        ],
        out_specs=pl.BlockSpec((BN, D), lambda i: (i, 0)),
        out_shape=jax.ShapeDtypeStruct((N, D), jnp.float32),
    )(partials, partials, xp, bias.reshape(1, D), gamma.reshape(1, D), beta.reshape(1, D))


def kernel(x, edge_index,
           Wl0, bl0, Wr0, br0, att0, bias0, gamma0, beta0,
           Wl1, bl1, Wr1, br1, att1, bias1, gamma1, beta1):
    src = edge_index[0].astype(jnp.int32)
    dst = edge_index[1].astype(jnp.int32)
    xp = x
    layers = (
        (Wl0, bl0, Wr0, br0, att0, bias0, gamma0, beta0),
        (Wl1, bl1, Wr1, br1, att1, bias1, gamma1, beta1),
    )
    for (Wl, bl, Wr, br, att, bias, gamma, beta) in layers:
        xl, xr = _linear(xp, Wl[:, _PERM], bl[_PERM], Wr[:, _PERM], br[_PERM])
        partials = _edge_pass(xl, xr, src, dst, att)
        xp = _finalize(partials, xp, bias, gamma, beta)
    return xp[:N]


# speedup vs baseline: 1.7992x; 1.0008x over previous
import jax
import jax.numpy as jnp
import numpy as np
from jax import lax
from jax.experimental import pallas as pl
from jax.experimental.pallas import tpu as pltpu
from jax.experimental.pallas import tpu_sc as plsc

N = 10000
E = 320000
D = 128
H = 8
C = 16
ACC_W = 144
EB = 40
NTILES = 32
EDGES_PER_TILE = E // NTILES
NUM_BATCHES = EDGES_PER_TILE // EB
ROWS_PER_TILE = N // 16

_PERM = np.empty((D,), np.int32)
for _p in range(H // 2):
    for _c in range(C):
        _PERM[32 * _p + 2 * _c] = (2 * _p) * 16 + _c
        _PERM[32 * _p + 2 * _c + 1] = (2 * _p + 1) * 16 + _c


def _lin_body(x_ref, wl_ref, bl_ref, wr_ref, br_ref, xl_ref, xr_ref):
    xb = x_ref[...]
    xl_ref[...] = (jnp.dot(xb, wl_ref[...], preferred_element_type=jnp.float32)
                   + bl_ref[...]).astype(jnp.bfloat16)
    xr_ref[...] = (jnp.dot(xb, wr_ref[...], preferred_element_type=jnp.float32)
                   + br_ref[...]).astype(jnp.bfloat16)


def _linear(xp, Wl, bl, Wr, br):
    BN = 1000
    return pl.pallas_call(
        _lin_body,
        grid=(N // BN,),
        in_specs=[
            pl.BlockSpec((BN, D), lambda i: (i, 0)),
            pl.BlockSpec((D, D), lambda i: (0, 0)),
            pl.BlockSpec((1, D), lambda i: (0, 0)),
            pl.BlockSpec((D, D), lambda i: (0, 0)),
            pl.BlockSpec((1, D), lambda i: (0, 0)),
        ],
        out_specs=[
            pl.BlockSpec((BN, D), lambda i: (i, 0)),
            pl.BlockSpec((BN, D), lambda i: (i, 0)),
        ],
        out_shape=[jax.ShapeDtypeStruct((N, D), jnp.bfloat16)] * 2,
    )(xp, Wl, bl.reshape(1, D), Wr, br.reshape(1, D))


_GDN = lax.GatherDimensionNumbers(
    offset_dims=(), collapsed_slice_dims=(0,), start_index_map=(0,))


def _lane_shuffle(v, perm):
    return lax.gather(v, perm[:, None], _GDN, slice_sizes=(1,),
                      mode=lax.GatherScatterMode.PROMISE_IN_BOUNDS)


def _sum_splat(v, perms):
    for perm in perms:
        v = v + _lane_shuffle(v, perm)
    return v


def _edge_body(xl_hbm, xr_hbm, src_hbm, dst_hbm, att_hbm, out_hbm,
               accum, srcg_a, srcg_b, dstg_a, dstg_b, dsts_a, dsts_b,
               xl_a, xr_a, xl_b, xr_b, orow_a, orow_b, att_v,
               sem_ga, sem_gb, sem_da, sem_db, sem_sa, sem_sb):
    c = lax.axis_index("c")
    s = lax.axis_index("s")
    wid = c * 16 + s
    base0 = wid * EDGES_PER_TILE

    pltpu.sync_copy(att_hbm, att_v)

    zero = jnp.zeros((16,), jnp.float32)

    def zrow(i, carry):
        for j in range(ACC_W // 16):
            orow_a[i, pl.ds(j * 16, 16)] = zero
            orow_b[i, pl.ds(j * 16, 16)] = zero
        return carry

    lax.fori_loop(0, EB, zrow, 0)
    for k in range(ROWS_PER_TILE // EB):
        pltpu.sync_copy(orow_a, accum.at[pl.ds(s * ROWS_PER_TILE + k * EB, EB)])
    rem = ROWS_PER_TILE % EB
    if rem:
        pltpu.sync_copy(
            orow_a.at[pl.ds(0, rem)],
            accum.at[pl.ds(s * ROWS_PER_TILE + (ROWS_PER_TILE // EB) * EB, rem)])
    plsc.subcore_barrier()

    att_regs = [att_v[h, :] for h in range(H)]
    lane = lax.iota(jnp.int32, 16)
    perm15 = lane * 0 + 15

    last = NUM_BATCHES - 1

    def issue_gidx(b, srcg, dstg, sem_d):
        pltpu.async_copy(src_hbm.at[pl.ds(base0 + b * EB, EB)], srcg, sem_d)
        pltpu.async_copy(dst_hbm.at[pl.ds(base0 + b * EB, EB)], dstg, sem_d)

    def wait_gidx(srcg, dstg, sem_d):
        pltpu.make_async_copy(src_hbm.at[pl.ds(0, EB)], srcg, sem_d).wait()
        pltpu.make_async_copy(dst_hbm.at[pl.ds(0, EB)], dstg, sem_d).wait()

    def issue_gathers(xl_buf, xr_buf, srcg, dstg, sem_g):
        pltpu.async_copy(xl_hbm.at[srcg], xl_buf, sem_g)
        pltpu.async_copy(xr_hbm.at[dstg], xr_buf, sem_g)

    def wait_gathers(xl_buf, xr_buf, sem_g):
        pltpu.make_async_copy(src_hbm.at[pl.ds(0, EB)], xl_buf, sem_g).wait()
        pltpu.make_async_copy(src_hbm.at[pl.ds(0, EB)], xr_buf, sem_g).wait()

    def vcopy_idx(src_ref, dst_ref):
        for off in (0, 16, EB - 16):
            dst_ref[pl.ds(off, 16)] = src_ref[pl.ds(off, 16)]

    def issue_scatter(orow, dsts, sem_s):
        pltpu.async_copy(orow, accum.at[dsts], sem_s, add=True)

    def wait_scatter(orow, dsts, sem_s):
        pltpu.make_async_copy(orow, accum.at[dsts], sem_s).wait()

    def compute(xl_buf, xr_buf, orow):
        @plsc.parallel_loop(0, EB, unroll=4)
        def edge_body(i):
            dv = jnp.zeros((16,), jnp.float32)
            for p in range(H // 2):
                xjs = plsc.unpack(xl_buf[i, pl.ds(p * 32, 32)],
                                  format=plsc.PackFormat.INTERLEAVED)
                xis = plsc.unpack(xr_buf[i, pl.ds(p * 32, 32)],
                                  format=plsc.PackFormat.INTERLEAVED)
                for q in (0, 1):
                    h = 2 * p + q
                    xj = xjs[q]
                    sv = xis[q] + xj
                    t = jnp.maximum(sv, 0.2 * sv)
                    alpha = _lane_shuffle(jnp.cumsum(t * att_regs[h]), perm15)
                    ev = jnp.exp(alpha)
                    orow[i, pl.ds(h * 16, 16)] = xj * ev
                    dv = jnp.where(lane == h, ev, dv)
            orow[i, pl.ds(D, 16)] = dv

    A = (xl_a, xr_a, srcg_a, dstg_a, dsts_a, orow_a, sem_ga, sem_da, sem_sa)
    B = (xl_b, xr_b, srcg_b, dstg_b, dsts_b, orow_b, sem_gb, sem_db, sem_sb)

    issue_gidx(0, srcg_a, dstg_a, sem_da)
    wait_gidx(srcg_a, dstg_a, sem_da)
    issue_gathers(xl_a, xr_a, srcg_a, dstg_a, sem_ga)
    issue_gidx(1, srcg_b, dstg_b, sem_db)
    vcopy_idx(dstg_a, dsts_a)
    vcopy_idx(dstg_a, dsts_b)
    issue_scatter(orow_a, dsts_a, sem_sa)
    issue_scatter(orow_b, dsts_b, sem_sb)

    def sub_iter(b, P, Q):
        xlP, xrP, srcgP, dstgP, dstsP, orowP, sem_gP, sem_dP, sem_sP = P
        xlQ, xrQ, srcgQ, dstgQ, dstsQ, orowQ, sem_gQ, sem_dQ, sem_sQ = Q
        wait_gidx(srcgQ, dstgQ, sem_dQ)
        issue_gathers(xlQ, xrQ, srcgQ, dstgQ, sem_gQ)
        wait_gathers(xlP, xrP, sem_gP)
        wait_scatter(orowP, dstsP, sem_sP)
        vcopy_idx(dstgP, dstsP)
        issue_gidx(jnp.minimum(b + 2, last), srcgP, dstgP, sem_dP)
        compute(xlP, xrP, orowP)
        issue_scatter(orowP, dstsP, sem_sP)

    def batch_pair(k, carry):
        sub_iter(2 * k, A, B)
        sub_iter(2 * k + 1, B, A)
        return carry

    lax.fori_loop(0, NUM_BATCHES // 2, batch_pair, 0)
    if NUM_BATCHES % 2:
        sub_iter(NUM_BATCHES - 1, A, B)
        wait_scatter(orow_a, dsts_a, sem_sa)
        wait_scatter(orow_b, dsts_b, sem_sb)
        wait_gathers(xl_b, xr_b, sem_gb)
        wait_gidx(srcg_a, dstg_a, sem_da)
    else:
        wait_scatter(orow_a, dsts_a, sem_sa)
        wait_scatter(orow_b, dsts_b, sem_sb)
        wait_gathers(xl_a, xr_a, sem_ga)
        wait_gidx(srcg_b, dstg_b, sem_db)
    plsc.subcore_barrier()

    pltpu.sync_copy(
        accum.at[pl.ds(s * ROWS_PER_TILE, ROWS_PER_TILE)],
        out_hbm.at[pl.ds(c * N + s * ROWS_PER_TILE, ROWS_PER_TILE)],
    )


def _edge_pass(xl, xr, src, dst, att):
    mesh = plsc.VectorSubcoreMesh(core_axis_name="c", subcore_axis_name="s")
    kfn = pl.kernel(
        _edge_body,
        out_type=jax.ShapeDtypeStruct((2 * N, ACC_W), jnp.float32),
        mesh=mesh,
        scratch_types=[
            pltpu.VMEM_SHARED((N, ACC_W), jnp.float32),
            pltpu.VMEM((EB,), jnp.int32),
            pltpu.VMEM((EB,), jnp.int32),
            pltpu.VMEM((EB,), jnp.int32),
            pltpu.VMEM((EB,), jnp.int32),
            pltpu.VMEM((EB,), jnp.int32),
            pltpu.VMEM((EB,), jnp.int32),
            pltpu.VMEM((EB, D), jnp.bfloat16),
            pltpu.VMEM((EB, D), jnp.bfloat16),
            pltpu.VMEM((EB, D), jnp.bfloat16),
            pltpu.VMEM((EB, D), jnp.bfloat16),
            pltpu.VMEM((EB, ACC_W), jnp.float32),
            pltpu.VMEM((EB, ACC_W), jnp.float32),
            pltpu.VMEM((H, C), jnp.float32),
            pltpu.SemaphoreType.DMA,
            pltpu.SemaphoreType.DMA,
            pltpu.SemaphoreType.DMA,
            pltpu.SemaphoreType.DMA,
            pltpu.SemaphoreType.DMA,
            pltpu.SemaphoreType.DMA,
        ],
        compiler_params=pltpu.CompilerParams(use_tc_tiling_on_sc=False,
                                             needs_layout_passes=False),
    )
    return kfn(xl, xr, src, dst, att)


def _final_body(p0_ref, p1_ref, x_ref, bias_ref, gamma_ref, beta_ref, o_ref):
    num = p0_ref[:, :D] + p1_ref[:, :D]
    den16 = p0_ref[:, D:] + p1_ref[:, D:]
    eh = lax.broadcasted_iota(jnp.int32, (16, D), 0)
    ej = lax.broadcasted_iota(jnp.int32, (16, D), 1) // C
    expand = (eh == ej).astype(jnp.float32)
    den = jnp.dot(den16, expand, preferred_element_type=jnp.float32)
    out = num / (den + 1e-16)
    hh = out + bias_ref[...]
    hh = jnp.where(hh > 0, hh, jnp.exp(jnp.minimum(hh, 0.0)) - 1.0)
    y = hh + x_ref[...]
    mu = jnp.mean(y, axis=-1, keepdims=True)
    var = jnp.mean((y - mu) ** 2, axis=-1, keepdims=True)
    o_ref[...] = gamma_ref[...] * (y - mu) / jnp.sqrt(var + 1e-5) + beta_ref[...]


def _finalize(partials, xp, bias, gamma, beta):
    BN = 1000
    nblk = N // BN
    return pl.pallas_call(
        _final_body,
        grid=(nblk,),
        in_specs=[
            pl.BlockSpec((BN, ACC_W), lambda i: (i, 0)),
            pl.BlockSpec((BN, ACC_W), lambda i, _n=nblk: (i + _n, 0)),
            pl.BlockSpec((BN, D), lambda i: (i, 0)),
            pl.BlockSpec((1, D), lambda i: (0, 0)),
            pl.BlockSpec((1, D), lambda i: (0, 0)),
            pl.BlockSpec((1, D), lambda i: (0, 0)),
        ],
        out_specs=pl.BlockSpec((BN, D), lambda i: (i, 0)),
        out_shape=jax.ShapeDtypeStruct((N, D), jnp.float32),
    )(partials, partials, xp, bias.reshape(1, D), gamma.reshape(1, D), beta.reshape(1, D))


def kernel(x, edge_index,
           Wl0, bl0, Wr0, br0, att0, bias0, gamma0, beta0,
           Wl1, bl1, Wr1, br1, att1, bias1, gamma1, beta1):
    src = edge_index[0].astype(jnp.int32)
    dst = edge_index[1].astype(jnp.int32)
    xp = x
    layers = (
        (Wl0, bl0, Wr0, br0, att0, bias0, gamma0, beta0),
        (Wl1, bl1, Wr1, br1, att1, bias1, gamma1, beta1),
    )
    for (Wl, bl, Wr, br, att, bias, gamma, beta) in layers:
        xl, xr = _linear(xp, Wl[:, _PERM], bl[_PERM], Wr[:, _PERM], br[_PERM])
        partials = _edge_pass(xl, xr, src, dst, att)
        xp = _finalize(partials, xp, bias, gamma, beta)
    return xp[:N]
